# baseline (speedup 1.0000x reference)
import functools

import jax
import jax.numpy as jnp
from jax import lax
from jax.experimental import pallas as pl
from jax.experimental.pallas import tpu as pltpu
from jax.experimental.pallas import tpu_sc as plsc

NC = 2
NS = 16
NW = NC * NS
LANES = 16
CHUNK = 128
ROWS = 1024
SPMEM_WORDS = 2**21 - 1

_UNTILED = pltpu.CompilerParams(use_tc_tiling_on_sc=False)
def _mesh():
    return plsc.VectorSubcoreMesh(core_axis_name="c", subcore_axis_name="s")


def _sc_degree(ep, n1, nch):
    rpt = n1 // NS

    @functools.partial(
        pl.kernel,
        out_type=jax.ShapeDtypeStruct((NC, n1), jnp.float32),
        mesh=_mesh(),
        compiler_params=_UNTILED,
        scratch_types=[
            pltpu.VMEM((nch, CHUNK), jnp.int32),
            pltpu.VMEM((CHUNK,), jnp.float32),
            pltpu.VMEM((rpt,), jnp.float32),
            pltpu.VMEM_SHARED((n1,), jnp.float32),
        ],
    )
    def k(ep_hbm, out_hbm, didx, ones_v, zbuf, deg_sh):
        cid = lax.axis_index("c")
        sid = lax.axis_index("s")
        wid = cid * NS + sid

        @pl.loop(0, CHUNK // LANES)
        def _(i):
            ones_v[pl.ds(i * LANES, LANES)] = jnp.full((LANES,), 1.0, jnp.float32)

        @pl.loop(0, rpt // LANES)
        def _(i):
            zbuf[pl.ds(i * LANES, LANES)] = jnp.zeros((LANES,), jnp.float32)

        pltpu.sync_copy(zbuf, deg_sh.at[pl.ds(sid * rpt, rpt)])
        pltpu.sync_copy(ep_hbm.at[1, wid], didx)
        plsc.subcore_barrier()

        @pl.loop(0, nch)
        def _(j):
            pltpu.sync_copy(ones_v, deg_sh.at[didx.at[j]], add=True)

        plsc.subcore_barrier()
        pltpu.sync_copy(deg_sh.at[pl.ds(sid * rpt, rpt)], zbuf)
        pltpu.sync_copy(zbuf, out_hbm.at[cid, pl.ds(sid * rpt, rpt)])

    return k(ep)


def _sc_aggregate(g, ep, n1, nch, d, split):
    rpt = n1 // NS
    nzc = rpt // CHUNK

    per_tile = (SPMEM_WORDS - 2 * n1 * d - 4096) // NS
    assert 2 * CHUNK * d + 2 * nch * CHUNK <= per_tile, "spmem budget exceeded"

    @functools.partial(
        pl.kernel,
        out_type=jax.ShapeDtypeStruct((NC, n1, d), jnp.float32),
        mesh=_mesh(),
        compiler_params=_UNTILED,
        scratch_types=[
            pltpu.VMEM((nch, CHUNK), jnp.int32),
            pltpu.VMEM((nch, CHUNK), jnp.int32),
            pltpu.VMEM((CHUNK, d), jnp.float32),
            pltpu.VMEM((CHUNK, d), jnp.float32),
            pltpu.VMEM_SHARED((n1, d), jnp.float32),
            pltpu.VMEM_SHARED((n1, d), jnp.float32),
            pltpu.SemaphoreType.DMA,
            pltpu.SemaphoreType.DMA,
        ],
    )
    def k(g_hbm, ep_hbm, out_hbm,
          sidx, didx, rbuf0, rbuf1, g_sh, acc_sh, sem0, sem1):
        cid = lax.axis_index("c")
        sid = lax.axis_index("s")
        row0 = sid * rpt

        if split:
            pltpu.sync_copy(g_hbm.at[cid, pl.ds(row0, rpt)], g_sh.at[pl.ds(row0, rpt)])
        else:
            pltpu.sync_copy(g_hbm.at[pl.ds(row0, rpt)], g_sh.at[pl.ds(row0, rpt)])

        zoffs = list(range(0, d - LANES + 1, LANES))
        if d % LANES:
            zoffs.append(d - LANES)

        @pl.loop(0, CHUNK)
        def _(i):
            for q in zoffs:
                rbuf0[i, pl.ds(q, LANES)] = jnp.zeros((LANES,), jnp.float32)

        @pl.loop(0, nzc)
        def _(kk):
            pltpu.sync_copy(rbuf0, acc_sh.at[pl.ds(row0 + kk * CHUNK, CHUNK)])

        plsc.subcore_barrier()

        npass = NC if split else 1

        @pl.loop(0, npass)
        def _(p):
            eid = NC * sid + p if split else cid * NS + sid
            pltpu.sync_copy(ep_hbm.at[0, eid], sidx)
            pltpu.sync_copy(ep_hbm.at[1, eid], didx)
            pltpu.async_copy(g_sh.at[sidx.at[0]], rbuf0, sem0)
            pltpu.async_copy(g_sh.at[sidx.at[1]], rbuf1, sem1)

            @pl.loop(0, nch, step=2)
            def _(j):
                pltpu.make_async_copy(g_sh.at[sidx.at[j]], rbuf0, sem0).wait()
                pltpu.sync_copy(rbuf0, acc_sh.at[didx.at[j]], add=True)

                @pl.when(j + 2 < nch)
                def _():
                    pltpu.async_copy(g_sh.at[sidx.at[j + 2]], rbuf0, sem0)

                pltpu.make_async_copy(g_sh.at[sidx.at[j + 1]], rbuf1, sem1).wait()
                pltpu.sync_copy(rbuf1, acc_sh.at[didx.at[j + 1]], add=True)

                @pl.when(j + 3 < nch)
                def _():
                    pltpu.async_copy(g_sh.at[sidx.at[j + 3]], rbuf1, sem1)

        plsc.subcore_barrier()
        pltpu.sync_copy(acc_sh.at[pl.ds(row0, rpt)], out_hbm.at[cid, pl.ds(row0, rpt)])

    return k(g, ep)


def _dinv_of(deg_ref):
    return lax.rsqrt(deg_ref[:, 0:1] + deg_ref[:, 1:2] + 1.0)


def _tc_dense1(x, w1, degt, n1, f_in, f_hid):
    hd = f_hid // NC

    def body(x_ref, w_ref, deg_ref, o_ref):
        dinv = _dinv_of(deg_ref)
        h = jnp.dot(x_ref[...], w_ref[...], preferred_element_type=jnp.float32)
        g = h * dinv
        o_ref[0] = g[:, :hd]
        o_ref[1] = g[:, hd:]

    return pl.pallas_call(
        body,
        grid=(n1 // ROWS,),
        in_specs=[
            pl.BlockSpec((ROWS, f_in), lambda i: (i, 0)),
            pl.BlockSpec((f_in, f_hid), lambda i: (0, 0)),
            pl.BlockSpec((ROWS, 2), lambda i: (i, 0)),
        ],
        out_specs=pl.BlockSpec((NC, ROWS, hd), lambda i: (0, i, 0)),
        out_shape=jax.ShapeDtypeStruct((NC, n1, hd), jnp.float32),
    )(x, w1, degt)


def _tc_dense2(acc1, g1, degt, w2p, n1, f_hid, d2):
    hd = f_hid // NC

    def body(a_ref, g_ref, deg_ref, w_ref, o_ref):
        dinv = _dinv_of(deg_ref)
        s = jnp.concatenate([a_ref[0] + g_ref[0], a_ref[1] + g_ref[1]], axis=1)
        z = jnp.maximum(s * dinv, 0.0)
        o_ref[...] = jnp.dot(z, w_ref[...], preferred_element_type=jnp.float32) * dinv

    return pl.pallas_call(
        body,
        grid=(n1 // ROWS,),
        in_specs=[
            pl.BlockSpec((NC, ROWS, hd), lambda i: (0, i, 0)),
            pl.BlockSpec((NC, ROWS, hd), lambda i: (0, i, 0)),
            pl.BlockSpec((ROWS, 2), lambda i: (i, 0)),
            pl.BlockSpec((f_hid, d2), lambda i: (0, 0)),
        ],
        out_specs=pl.BlockSpec((ROWS, d2), lambda i: (i, 0)),
        out_shape=jax.ShapeDtypeStruct((n1, d2), jnp.float32),
    )(acc1, g1, degt, w2p)


def _tc_dense3(acc2, g2, degt, n, n1, d2, f_out):
    def body(a_ref, g_ref, deg_ref, o_ref):
        dinv = _dinv_of(deg_ref)
        v = (a_ref[0] + a_ref[1] + g_ref[...]) * dinv
        o_ref[...] = v[:, :f_out]

    return pl.pallas_call(
        body,
        grid=(n1 // ROWS,),
        in_specs=[
            pl.BlockSpec((NC, ROWS, d2), lambda i: (0, i, 0)),
            pl.BlockSpec((ROWS, d2), lambda i: (i, 0)),
            pl.BlockSpec((ROWS, 2), lambda i: (i, 0)),
        ],
        out_specs=pl.BlockSpec((ROWS, f_out), lambda i: (i, 0)),
        out_shape=jax.ShapeDtypeStruct((n, f_out), jnp.float32),
    )(acc2, g2, degt)


def kernel(x, edge_index, W1, W2):
    n, f_in = x.shape
    f_hid = W1.shape[1]
    f_out = W2.shape[1]
    e = edge_index.shape[1]

    ept = NW * CHUNK
    nch = -(-e // ept)
    if nch % 2:
        nch += 1
    e_pad = nch * ept
    n1 = -(-(n + 2) // (NS * CHUNK)) * (NS * CHUNK)
    d2 = max(LANES, -(-f_out // 8) * 8)

    pads = jnp.stack([jnp.zeros((e_pad - e,), jnp.int32),
                      jnp.full((e_pad - e,), n, jnp.int32)])
    ep = jnp.concatenate([edge_index, pads], axis=1).reshape(2, NW, nch, CHUNK)

    w2p = W2 if d2 == f_out else jnp.pad(W2, ((0, 0), (0, d2 - f_out)))

    deg2 = _sc_degree(ep, n1, nch)
    degt = deg2.T.reshape(n1, NC)

    g1 = _tc_dense1(x, W1, degt, n1, f_in, f_hid)
    acc1 = _sc_aggregate(g1, ep, n1, nch, f_hid // NC, split=True)
    g2 = _tc_dense2(acc1, g1, degt, w2p, n1, f_hid, d2)
    acc2 = _sc_aggregate(g2, ep, n1, nch, d2, split=False)
    return _tc_dense3(acc2, g2, degt, n, n1, d2, f_out)


# speedup vs baseline: 1.0089x; 1.0089x over previous
import functools

import jax
import jax.numpy as jnp
from jax import lax
from jax.experimental import pallas as pl
from jax.experimental.pallas import tpu as pltpu
from jax.experimental.pallas import tpu_sc as plsc

NC = 2
NS = 16
NW = NC * NS
LANES = 16
CHUNK = 128
ROWS = 2048
SPMEM_WORDS = 2**21 - 1

_UNTILED = pltpu.CompilerParams(use_tc_tiling_on_sc=False)
_UNTILED_NL = pltpu.CompilerParams(
    use_tc_tiling_on_sc=False, needs_layout_passes=False)
def _mesh():
    return plsc.VectorSubcoreMesh(core_axis_name="c", subcore_axis_name="s")


def _sc_degree(ep, n1, nch):
    rpt = n1 // NS

    @functools.partial(
        pl.kernel,
        out_type=jax.ShapeDtypeStruct((NC, n1), jnp.float32),
        mesh=_mesh(),
        compiler_params=_UNTILED_NL,
        scratch_types=[
            pltpu.VMEM((nch, CHUNK), jnp.int32),
            pltpu.VMEM((n1,), jnp.float32),
            pltpu.VMEM((rpt,), jnp.float32),
            pltpu.VMEM_SHARED((NS, n1), jnp.float32),
        ],
    )
    def k(ep_hbm, out_hbm, didx, deg_l, tbuf, stage_sh):
        cid = lax.axis_index("c")
        sid = lax.axis_index("s")
        wid = cid * NS + sid
        row0 = sid * rpt
        ones = jnp.full((LANES,), 1.0, jnp.float32)

        pltpu.sync_copy(ep_hbm.at[1, wid], didx)

        @pl.loop(0, n1 // LANES, unroll=4)
        def _(i):
            deg_l[pl.ds(i * LANES, LANES)] = jnp.zeros((LANES,), jnp.float32)

        @pl.loop(0, nch)
        def _(j):
            for q in range(CHUNK // LANES):
                idx = didx[j, pl.ds(q * LANES, LANES)]
                plsc.addupdate_scatter(deg_l, [idx], ones)

        pltpu.sync_copy(deg_l, stage_sh.at[sid])
        plsc.subcore_barrier()
        pltpu.sync_copy(stage_sh.at[0, pl.ds(row0, rpt)], deg_l.at[pl.ds(0, rpt)])

        @pl.loop(1, NS)
        def _(t):
            pltpu.sync_copy(stage_sh.at[t, pl.ds(row0, rpt)], tbuf)

            @pl.loop(0, rpt // LANES, unroll=4)
            def _(i):
                sl = pl.ds(i * LANES, LANES)
                deg_l[sl] = deg_l[sl] + tbuf[sl]

        pltpu.sync_copy(deg_l.at[pl.ds(0, rpt)], out_hbm.at[cid, pl.ds(row0, rpt)])

    return k(ep)


def _sc_aggregate(g, ep, n1, nch, d, split):
    rpt = n1 // NS
    nzc = rpt // CHUNK

    per_tile = (SPMEM_WORDS - 2 * n1 * d - 4096) // NS
    assert 2 * CHUNK * d + 2 * nch * CHUNK <= per_tile, "spmem budget exceeded"

    @functools.partial(
        pl.kernel,
        out_type=jax.ShapeDtypeStruct((NC, n1, d), jnp.float32),
        mesh=_mesh(),
        compiler_params=_UNTILED,
        scratch_types=[
            pltpu.VMEM((nch, CHUNK), jnp.int32),
            pltpu.VMEM((nch, CHUNK), jnp.int32),
            pltpu.VMEM((CHUNK, d), jnp.float32),
            pltpu.VMEM((CHUNK, d), jnp.float32),
            pltpu.VMEM_SHARED((n1, d), jnp.float32),
            pltpu.VMEM_SHARED((n1, d), jnp.float32),
            pltpu.SemaphoreType.DMA,
            pltpu.SemaphoreType.DMA,
        ],
    )
    def k(g_hbm, ep_hbm, out_hbm,
          sidx, didx, rbuf0, rbuf1, g_sh, acc_sh, sem0, sem1):
        cid = lax.axis_index("c")
        sid = lax.axis_index("s")
        row0 = sid * rpt

        if split:
            pltpu.sync_copy(g_hbm.at[cid, pl.ds(row0, rpt)], g_sh.at[pl.ds(row0, rpt)])
        else:
            pltpu.sync_copy(g_hbm.at[pl.ds(row0, rpt)], g_sh.at[pl.ds(row0, rpt)])

        zoffs = list(range(0, d - LANES + 1, LANES))
        if d % LANES:
            zoffs.append(d - LANES)

        @pl.loop(0, CHUNK)
        def _(i):
            for q in zoffs:
                rbuf0[i, pl.ds(q, LANES)] = jnp.zeros((LANES,), jnp.float32)

        @pl.loop(0, nzc)
        def _(kk):
            pltpu.sync_copy(rbuf0, acc_sh.at[pl.ds(row0 + kk * CHUNK, CHUNK)])

        plsc.subcore_barrier()

        npass = NC if split else 1

        @pl.loop(0, npass)
        def _(p):
            eid = NC * sid + p if split else cid * NS + sid
            pltpu.sync_copy(ep_hbm.at[0, eid], sidx)
            pltpu.sync_copy(ep_hbm.at[1, eid], didx)
            pltpu.async_copy(g_sh.at[sidx.at[0]], rbuf0, sem0)
            pltpu.async_copy(g_sh.at[sidx.at[1]], rbuf1, sem1)

            @pl.loop(0, nch, step=2)
            def _(j):
                pltpu.make_async_copy(g_sh.at[sidx.at[j]], rbuf0, sem0).wait()
                pltpu.sync_copy(rbuf0, acc_sh.at[didx.at[j]], add=True)

                @pl.when(j + 2 < nch)
                def _():
                    pltpu.async_copy(g_sh.at[sidx.at[j + 2]], rbuf0, sem0)

                pltpu.make_async_copy(g_sh.at[sidx.at[j + 1]], rbuf1, sem1).wait()
                pltpu.sync_copy(rbuf1, acc_sh.at[didx.at[j + 1]], add=True)

                @pl.when(j + 3 < nch)
                def _():
                    pltpu.async_copy(g_sh.at[sidx.at[j + 3]], rbuf1, sem1)

        plsc.subcore_barrier()
        pltpu.sync_copy(acc_sh.at[pl.ds(row0, rpt)], out_hbm.at[cid, pl.ds(row0, rpt)])

    return k(g, ep)


def _dinv_of(deg_ref):
    return lax.rsqrt(deg_ref[:, 0:1] + deg_ref[:, 1:2] + 1.0)


def _tc_dense1(x, w1, degt, n1, f_in, f_hid):
    hd = f_hid // NC

    def body(x_ref, w_ref, deg_ref, o_ref):
        dinv = _dinv_of(deg_ref)
        h = jnp.dot(x_ref[...], w_ref[...], preferred_element_type=jnp.float32)
        g = h * dinv
        o_ref[0] = g[:, :hd]
        o_ref[1] = g[:, hd:]

    return pl.pallas_call(
        body,
        grid=(n1 // ROWS,),
        in_specs=[
            pl.BlockSpec((ROWS, f_in), lambda i: (i, 0)),
            pl.BlockSpec((f_in, f_hid), lambda i: (0, 0)),
            pl.BlockSpec((ROWS, 2), lambda i: (i, 0)),
        ],
        out_specs=pl.BlockSpec((NC, ROWS, hd), lambda i: (0, i, 0)),
        out_shape=jax.ShapeDtypeStruct((NC, n1, hd), jnp.float32),
    )(x, w1, degt)


def _tc_dense2(acc1, g1, degt, w2p, n1, f_hid, d2):
    hd = f_hid // NC

    def body(a_ref, g_ref, deg_ref, w_ref, o_ref):
        dinv = _dinv_of(deg_ref)
        s = jnp.concatenate([a_ref[0] + g_ref[0], a_ref[1] + g_ref[1]], axis=1)
        z = jnp.maximum(s * dinv, 0.0)
        o_ref[...] = jnp.dot(z, w_ref[...], preferred_element_type=jnp.float32) * dinv

    return pl.pallas_call(
        body,
        grid=(n1 // ROWS,),
        in_specs=[
            pl.BlockSpec((NC, ROWS, hd), lambda i: (0, i, 0)),
            pl.BlockSpec((NC, ROWS, hd), lambda i: (0, i, 0)),
            pl.BlockSpec((ROWS, 2), lambda i: (i, 0)),
            pl.BlockSpec((f_hid, d2), lambda i: (0, 0)),
        ],
        out_specs=pl.BlockSpec((ROWS, d2), lambda i: (i, 0)),
        out_shape=jax.ShapeDtypeStruct((n1, d2), jnp.float32),
    )(acc1, g1, degt, w2p)


def _tc_dense3(acc2, g2, degt, n, n1, d2, f_out):
    def body(a_ref, g_ref, deg_ref, o_ref):
        dinv = _dinv_of(deg_ref)
        v = (a_ref[0] + a_ref[1] + g_ref[...]) * dinv
        o_ref[...] = v[:, :f_out]

    return pl.pallas_call(
        body,
        grid=(n1 // ROWS,),
        in_specs=[
            pl.BlockSpec((NC, ROWS, d2), lambda i: (0, i, 0)),
            pl.BlockSpec((ROWS, d2), lambda i: (i, 0)),
            pl.BlockSpec((ROWS, 2), lambda i: (i, 0)),
        ],
        out_specs=pl.BlockSpec((ROWS, f_out), lambda i: (i, 0)),
        out_shape=jax.ShapeDtypeStruct((n, f_out), jnp.float32),
    )(acc2, g2, degt)


def kernel(x, edge_index, W1, W2):
    n, f_in = x.shape
    f_hid = W1.shape[1]
    f_out = W2.shape[1]
    e = edge_index.shape[1]

    ept = NW * CHUNK
    nch = -(-e // ept)
    if nch % 2:
        nch += 1
    e_pad = nch * ept
    n1 = -(-(n + 2) // (NS * CHUNK)) * (NS * CHUNK)
    d2 = max(LANES, -(-f_out // 8) * 8)

    pads = jnp.stack([jnp.zeros((e_pad - e,), jnp.int32),
                      jnp.full((e_pad - e,), n, jnp.int32)])
    ep = jnp.concatenate([edge_index, pads], axis=1).reshape(2, NW, nch, CHUNK)

    w2p = W2 if d2 == f_out else jnp.pad(W2, ((0, 0), (0, d2 - f_out)))

    deg2 = _sc_degree(ep, n1, nch)
    degt = deg2.T.reshape(n1, NC)

    g1 = _tc_dense1(x, W1, degt, n1, f_in, f_hid)
    acc1 = _sc_aggregate(g1, ep, n1, nch, f_hid // NC, split=True)
    g2 = _tc_dense2(acc1, g1, degt, w2p, n1, f_hid, d2)
    acc2 = _sc_aggregate(g2, ep, n1, nch, d2, split=False)
    return _tc_dense3(acc2, g2, degt, n, n1, d2, f_out)


# speedup vs baseline: 1.0199x; 1.0109x over previous
import functools

import jax
import jax.numpy as jnp
from jax import lax
from jax.experimental import pallas as pl
from jax.experimental.pallas import tpu as pltpu
from jax.experimental.pallas import tpu_sc as plsc

NC = 2
NS = 16
NW = NC * NS
LANES = 16
CHUNK = 128
ROWS = 2048
SPMEM_WORDS = 2**21 - 1

_UNTILED = pltpu.CompilerParams(use_tc_tiling_on_sc=False)
def _mesh():
    return plsc.VectorSubcoreMesh(core_axis_name="c", subcore_axis_name="s")


def _sc_degree(ep, n1, nch):
    rpt = n1 // NS

    @functools.partial(
        pl.kernel,
        out_type=jax.ShapeDtypeStruct((NC, n1), jnp.float32),
        mesh=_mesh(),
        compiler_params=_UNTILED,
        scratch_types=[
            pltpu.VMEM((nch, CHUNK), jnp.int32),
            pltpu.VMEM((CHUNK,), jnp.float32),
            pltpu.VMEM((rpt,), jnp.float32),
            pltpu.VMEM_SHARED((n1,), jnp.float32),
        ],
    )
    def k(ep_hbm, out_hbm, didx, ones_v, zbuf, deg_sh):
        cid = lax.axis_index("c")
        sid = lax.axis_index("s")
        wid = cid * NS + sid

        @pl.loop(0, CHUNK // LANES)
        def _(i):
            ones_v[pl.ds(i * LANES, LANES)] = jnp.full((LANES,), 1.0, jnp.float32)

        @pl.loop(0, rpt // LANES)
        def _(i):
            zbuf[pl.ds(i * LANES, LANES)] = jnp.zeros((LANES,), jnp.float32)

        pltpu.sync_copy(zbuf, deg_sh.at[pl.ds(sid * rpt, rpt)])
        pltpu.sync_copy(ep_hbm.at[1, wid], didx)
        plsc.subcore_barrier()

        @pl.loop(0, nch)
        def _(j):
            pltpu.sync_copy(ones_v, deg_sh.at[didx.at[j]], add=True)

        plsc.subcore_barrier()
        pltpu.sync_copy(deg_sh.at[pl.ds(sid * rpt, rpt)], zbuf)
        pltpu.sync_copy(zbuf, out_hbm.at[cid, pl.ds(sid * rpt, rpt)])

    return k(ep)


def _sc_aggregate(g, ep, n1, nch, d, split):
    rpt = n1 // NS
    nzc = rpt // CHUNK

    per_tile = (SPMEM_WORDS - 2 * n1 * d - 4096) // NS
    assert 2 * CHUNK * d + 2 * nch * CHUNK <= per_tile, "spmem budget exceeded"

    @functools.partial(
        pl.kernel,
        out_type=jax.ShapeDtypeStruct((NC, n1, d), jnp.float32),
        mesh=_mesh(),
        compiler_params=_UNTILED,
        scratch_types=[
            pltpu.VMEM((nch, CHUNK), jnp.int32),
            pltpu.VMEM((nch, CHUNK), jnp.int32),
            pltpu.VMEM((CHUNK, d), jnp.float32),
            pltpu.VMEM((CHUNK, d), jnp.float32),
            pltpu.VMEM_SHARED((n1, d), jnp.float32),
            pltpu.VMEM_SHARED((n1, d), jnp.float32),
            pltpu.SemaphoreType.DMA,
            pltpu.SemaphoreType.DMA,
        ],
    )
    def k(g_hbm, ep_hbm, out_hbm,
          sidx, didx, rbuf0, rbuf1, g_sh, acc_sh, sem0, sem1):
        cid = lax.axis_index("c")
        sid = lax.axis_index("s")
        row0 = sid * rpt

        if split:
            pltpu.sync_copy(g_hbm.at[cid, pl.ds(row0, rpt)], g_sh.at[pl.ds(row0, rpt)])
        else:
            pltpu.sync_copy(g_hbm.at[pl.ds(row0, rpt)], g_sh.at[pl.ds(row0, rpt)])

        zoffs = list(range(0, d - LANES + 1, LANES))
        if d % LANES:
            zoffs.append(d - LANES)

        @pl.loop(0, CHUNK)
        def _(i):
            for q in zoffs:
                rbuf0[i, pl.ds(q, LANES)] = jnp.zeros((LANES,), jnp.float32)

        @pl.loop(0, nzc)
        def _(kk):
            pltpu.sync_copy(rbuf0, acc_sh.at[pl.ds(row0 + kk * CHUNK, CHUNK)])

        plsc.subcore_barrier()

        npass = NC if split else 1

        @pl.loop(0, npass)
        def _(p):
            eid = NC * sid + p if split else cid * NS + sid
            pltpu.sync_copy(ep_hbm.at[0, eid], sidx)
            pltpu.sync_copy(ep_hbm.at[1, eid], didx)
            pltpu.async_copy(g_sh.at[sidx.at[0]], rbuf0, sem0)
            pltpu.async_copy(g_sh.at[sidx.at[1]], rbuf1, sem1)

            @pl.loop(0, nch, step=2)
            def _(j):
                pltpu.make_async_copy(g_sh.at[sidx.at[j]], rbuf0, sem0).wait()
                pltpu.sync_copy(rbuf0, acc_sh.at[didx.at[j]], add=True)

                @pl.when(j + 2 < nch)
                def _():
                    pltpu.async_copy(g_sh.at[sidx.at[j + 2]], rbuf0, sem0)

                pltpu.make_async_copy(g_sh.at[sidx.at[j + 1]], rbuf1, sem1).wait()
                pltpu.sync_copy(rbuf1, acc_sh.at[didx.at[j + 1]], add=True)

                @pl.when(j + 3 < nch)
                def _():
                    pltpu.async_copy(g_sh.at[sidx.at[j + 3]], rbuf1, sem1)

        plsc.subcore_barrier()
        pltpu.sync_copy(acc_sh.at[pl.ds(row0, rpt)], out_hbm.at[cid, pl.ds(row0, rpt)])

    return k(g, ep)


def _dinv_of(deg_ref):
    return lax.rsqrt(deg_ref[:, 0:1] + deg_ref[:, 1:2] + 1.0)


def _tc_dense1(x, w1, degt, n1, f_in, f_hid):
    hd = f_hid // NC

    def body(x_ref, w_ref, deg_ref, o_ref):
        dinv = _dinv_of(deg_ref)
        h = jnp.dot(x_ref[...], w_ref[...], preferred_element_type=jnp.float32)
        g = h * dinv
        o_ref[0] = g[:, :hd]
        o_ref[1] = g[:, hd:]

    return pl.pallas_call(
        body,
        grid=(n1 // ROWS,),
        in_specs=[
            pl.BlockSpec((ROWS, f_in), lambda i: (i, 0)),
            pl.BlockSpec((f_in, f_hid), lambda i: (0, 0)),
            pl.BlockSpec((ROWS, 2), lambda i: (i, 0)),
        ],
        out_specs=pl.BlockSpec((NC, ROWS, hd), lambda i: (0, i, 0)),
        out_shape=jax.ShapeDtypeStruct((NC, n1, hd), jnp.float32),
    )(x, w1, degt)


def _tc_dense2(acc1, g1, degt, w2p, n1, f_hid, d2):
    hd = f_hid // NC

    def body(a_ref, g_ref, deg_ref, w_ref, o_ref):
        dinv = _dinv_of(deg_ref)
        s = jnp.concatenate([a_ref[0] + g_ref[0], a_ref[1] + g_ref[1]], axis=1)
        z = jnp.maximum(s * dinv, 0.0)
        o_ref[...] = jnp.dot(z, w_ref[...], preferred_element_type=jnp.float32) * dinv

    return pl.pallas_call(
        body,
        grid=(n1 // ROWS,),
        in_specs=[
            pl.BlockSpec((NC, ROWS, hd), lambda i: (0, i, 0)),
            pl.BlockSpec((NC, ROWS, hd), lambda i: (0, i, 0)),
            pl.BlockSpec((ROWS, 2), lambda i: (i, 0)),
            pl.BlockSpec((f_hid, d2), lambda i: (0, 0)),
        ],
        out_specs=pl.BlockSpec((ROWS, d2), lambda i: (i, 0)),
        out_shape=jax.ShapeDtypeStruct((n1, d2), jnp.float32),
    )(acc1, g1, degt, w2p)


def _tc_dense3(acc2, g2, degt, n, n1, d2, f_out):
    def body(a_ref, g_ref, deg_ref, o_ref):
        dinv = _dinv_of(deg_ref)
        v = (a_ref[0] + a_ref[1] + g_ref[...]) * dinv
        o_ref[...] = v[:, :f_out]

    return pl.pallas_call(
        body,
        grid=(n1 // ROWS,),
        in_specs=[
            pl.BlockSpec((NC, ROWS, d2), lambda i: (0, i, 0)),
            pl.BlockSpec((ROWS, d2), lambda i: (i, 0)),
            pl.BlockSpec((ROWS, 2), lambda i: (i, 0)),
        ],
        out_specs=pl.BlockSpec((ROWS, f_out), lambda i: (i, 0)),
        out_shape=jax.ShapeDtypeStruct((n, f_out), jnp.float32),
    )(acc2, g2, degt)


def kernel(x, edge_index, W1, W2):
    n, f_in = x.shape
    f_hid = W1.shape[1]
    f_out = W2.shape[1]
    e = edge_index.shape[1]

    ept = NW * CHUNK
    nch = -(-e // ept)
    if nch % 2:
        nch += 1
    e_pad = nch * ept
    n1 = -(-(n + 2) // (NS * CHUNK)) * (NS * CHUNK)
    d2 = max(LANES, -(-f_out // 8) * 8)

    pads = jnp.stack([jnp.zeros((e_pad - e,), jnp.int32),
                      jnp.full((e_pad - e,), n, jnp.int32)])
    ep = jnp.concatenate([edge_index, pads], axis=1).reshape(2, NW, nch, CHUNK)

    w2p = W2 if d2 == f_out else jnp.pad(W2, ((0, 0), (0, d2 - f_out)))

    deg2 = _sc_degree(ep, n1, nch)
    degt = deg2.T.reshape(n1, NC)

    g1 = _tc_dense1(x, W1, degt, n1, f_in, f_hid)
    acc1 = _sc_aggregate(g1, ep, n1, nch, f_hid // NC, split=True)
    g2 = _tc_dense2(acc1, g1, degt, w2p, n1, f_hid, d2)
    acc2 = _sc_aggregate(g2, ep, n1, nch, d2, split=False)
    return _tc_dense3(acc2, g2, degt, n, n1, d2, f_out)
